# baseline (speedup 1.0000x reference)
import functools

import jax
import jax.numpy as jnp
from jax.experimental import pallas as pl
from jax.experimental.pallas import tpu as pltpu


def _ohem_body(x_ref, t_ref, out_ref, m_ref, s_ref, tx_ref, ts_ref, *,
               batch: int):
    b = pl.program_id(0)
    j = pl.program_id(1)
    nj = pl.num_programs(1)

    @pl.when(j == 0)
    def _init():
        m_ref[0] = -jnp.inf
        s_ref[0] = 0.0
        tx_ref[0] = 0.0
        ts_ref[0] = 0.0

    x = x_ref[0]
    t = t_ref[0]

    m_old = m_ref[0]
    m_new = jnp.maximum(m_old, jnp.max(x))
    s_ref[0] = s_ref[0] * jnp.exp(m_old - m_new) + jnp.sum(jnp.exp(x - m_new))
    m_ref[0] = m_new
    tx_ref[0] += jnp.sum(t * x)
    ts_ref[0] += jnp.sum(t)

    @pl.when(j == nj - 1)
    def _finish():
        loss = ts_ref[0] * (m_ref[0] + jnp.log(s_ref[0])) - tx_ref[0]

        @pl.when(b == 0)
        def _zero():
            out_ref[0, 0] = 0.0

        out_ref[0, 0] += loss * (1.0 / batch)


def kernel(inputs, targets):
    batch = inputs.shape[0]
    n = inputs.shape[1] * inputs.shape[2] * inputs.shape[3]
    lanes = 128
    rows = n // lanes
    nchunk = 1
    for cand in (8, 7, 6, 4, 3, 2):
        if rows % cand == 0:
            nchunk = cand
            break
    chunk = rows // nchunk

    x = inputs.reshape(batch, rows, lanes)
    t = targets.reshape(batch, rows, lanes)

    out = pl.pallas_call(
        functools.partial(_ohem_body, batch=batch),
        grid=(batch, nchunk),
        in_specs=[
            pl.BlockSpec((1, chunk, lanes), lambda b, j: (b, j, 0)),
            pl.BlockSpec((1, chunk, lanes), lambda b, j: (b, j, 0)),
        ],
        out_specs=pl.BlockSpec((1, 1), lambda b, j: (0, 0),
                               memory_space=pltpu.SMEM),
        out_shape=jax.ShapeDtypeStruct((1, 1), jnp.float32),
        scratch_shapes=[
            pltpu.SMEM((1,), jnp.float32),
            pltpu.SMEM((1,), jnp.float32),
            pltpu.SMEM((1,), jnp.float32),
            pltpu.SMEM((1,), jnp.float32),
        ],
    )(x, t)
    return out[0, 0]


# speedup vs baseline: 1.0280x; 1.0280x over previous
import functools

import jax
import jax.numpy as jnp
from jax.experimental import pallas as pl
from jax.experimental.pallas import tpu as pltpu


def _ohem_body(x_ref, t_ref, out_ref, m_ref, s_ref, tx_ref, ts_ref, *,
               batch: int, chunk: int):
    b = pl.program_id(0)
    j = pl.program_id(1)
    nj = pl.num_programs(1)

    @pl.when(j == 0)
    def _init():
        m_ref[...] = jnp.full((8, 128), -jnp.inf, jnp.float32)
        s_ref[...] = jnp.zeros((8, 128), jnp.float32)
        tx_ref[...] = jnp.zeros((8, 128), jnp.float32)
        ts_ref[...] = jnp.zeros((8, 128), jnp.float32)

    x = x_ref[0].reshape(chunk // 8, 8, 128)
    t = t_ref[0].reshape(chunk // 8, 8, 128)

    m_old = m_ref[...]
    m_new = jnp.maximum(m_old, jnp.max(x, axis=0))
    s_ref[...] = (s_ref[...] * jnp.exp(m_old - m_new)
                  + jnp.sum(jnp.exp(x - m_new[None]), axis=0))
    m_ref[...] = m_new
    tx_ref[...] += jnp.sum(t * x, axis=0)
    ts_ref[...] += jnp.sum(t, axis=0)

    @pl.when(j == nj - 1)
    def _finish():
        m_vec = m_ref[...]
        m_glob = jnp.max(m_vec)
        s_tot = jnp.sum(s_ref[...] * jnp.exp(m_vec - m_glob))
        loss = (jnp.sum(ts_ref[...]) * (m_glob + jnp.log(s_tot))
                - jnp.sum(tx_ref[...]))

        @pl.when(b == 0)
        def _zero():
            out_ref[0, 0] = 0.0

        out_ref[0, 0] += loss * (1.0 / batch)


def kernel(inputs, targets):
    batch = inputs.shape[0]
    n = inputs.shape[1] * inputs.shape[2] * inputs.shape[3]
    lanes = 128
    rows = n // lanes
    nchunk = 1
    for cand in (8, 7, 6, 4, 3, 2):
        if rows % cand == 0:
            nchunk = cand
            break
    chunk = rows // nchunk

    x = inputs.reshape(batch, rows, lanes)
    t = targets.reshape(batch, rows, lanes)

    out = pl.pallas_call(
        functools.partial(_ohem_body, batch=batch, chunk=chunk),
        grid=(batch, nchunk),
        in_specs=[
            pl.BlockSpec((1, chunk, lanes), lambda b, j: (b, j, 0)),
            pl.BlockSpec((1, chunk, lanes), lambda b, j: (b, j, 0)),
        ],
        out_specs=pl.BlockSpec((1, 1), lambda b, j: (0, 0),
                               memory_space=pltpu.SMEM),
        out_shape=jax.ShapeDtypeStruct((1, 1), jnp.float32),
        scratch_shapes=[
            pltpu.VMEM((8, 128), jnp.float32),
            pltpu.VMEM((8, 128), jnp.float32),
            pltpu.VMEM((8, 128), jnp.float32),
            pltpu.VMEM((8, 128), jnp.float32),
        ],
    )(x, t)
    return out[0, 0]
